# Initial kernel scaffold; baseline (speedup 1.0000x reference)
import jax
import jax.numpy as jnp
from jax.experimental import pallas as pl


def kernel(x, edge_index, state, prev_edge, Wx, bx, Wh, bh):
    raise NotImplementedError("write your pallas kernel here")


# speedup vs baseline: 11.1068x; 11.1068x over previous
import functools

import jax
import jax.numpy as jnp
from jax import lax
from jax.experimental import pallas as pl
from jax.experimental.pallas import tpu as pltpu
from jax.experimental.pallas import tpu_sc as plsc

NC = 2
NS = 16
CHUNK = 128


def _dense_body(aggx_ref, aggh_ref, wx_ref, wh_ref, b_ref, c_ref, out_ref):
    g = b_ref[0, 0:1, :].astype(jnp.float32)
    g = g + jnp.dot(aggx_ref[0], wx_ref[0, 0], preferred_element_type=jnp.float32)
    g = g + jnp.dot(aggx_ref[1], wx_ref[0, 1], preferred_element_type=jnp.float32)
    g = g + jnp.dot(aggh_ref[0], wh_ref[0, 0], preferred_element_type=jnp.float32)
    g = g + jnp.dot(aggh_ref[1], wh_ref[0, 1], preferred_element_type=jnp.float32)
    H = 256
    f_t = jax.nn.sigmoid(g[:, 0:H])
    i_t = jax.nn.sigmoid(g[:, H:2 * H])
    o_t = jax.nn.sigmoid(g[:, 2 * H:3 * H])
    z_t = jnp.tanh(g[:, 3 * H:4 * H])
    c_prev = c_ref[0]
    c_t = f_t * c_prev + i_t * z_t
    h_t = o_t * c_t
    out_ref[0] = jnp.concatenate([h_t, c_t, f_t], axis=-1)


def kernel(x, edge_index, state, prev_edge, Wx, bx, Wh, bh):
    n, d = x.shape
    n_layers = Wx.shape[0]
    h = Wx.shape[3]
    e = edge_index.shape[1]

    per_w_chunks = (-(-e // (NS * CHUNK)) + 7) // 8 * 8
    e_pad = NS * per_w_chunks * CHUNK
    pad = e_pad - e

    def pad_edges(ei):
        src = jnp.concatenate([ei[0], jnp.zeros((pad,), jnp.int32)])
        dst = jnp.concatenate([ei[1], jnp.full((pad,), n, jnp.int32)])
        return src.reshape(e_pad // CHUNK, CHUNK), dst.reshape(e_pad // CHUNK, CHUNK)

    srcx, dstx = pad_edges(edge_index)
    srch, dsth = pad_edges(prev_edge)

    xflat = x.reshape(n * (d // CHUNK), CHUNK)
    stateflat = state.reshape(n_layers * n * (state.shape[2] // CHUNK), CHUNK)

    agg_flat = _sc_aggregate_all(xflat, stateflat, srcx, dstx, srch, dsth,
                                 n, per_w_chunks, n_layers)
    region = (-(-(n + 1) // NS) + 7) // 8 * 8 * NS
    agg6 = agg_flat.reshape((1 + n_layers) * NC, region, CHUNK)

    wxc = jnp.transpose(Wx, (0, 2, 1, 3)).reshape(n_layers, d, 4 * h)
    wxc = wxc.reshape(n_layers, d // CHUNK, CHUNK, 4 * h)
    whc = jnp.transpose(Wh, (0, 2, 1, 3)).reshape(n_layers, h, 4 * h)
    whc = whc.reshape(n_layers, h // CHUNK, CHUNK, 4 * h)
    bsum = (bx + bh).reshape(n_layers, 4 * h)
    bsum8 = jnp.broadcast_to(bsum[:, None, :], (n_layers, 8, 4 * h))

    nb = 1000
    grid = (n_layers, n // nb)
    out = pl.pallas_call(
        _dense_body,
        grid=grid,
        in_specs=[
            pl.BlockSpec((2, nb, CHUNK), lambda i, j: (0, j, 0)),
            pl.BlockSpec((2, nb, CHUNK), lambda i, j: (1 + i, j, 0)),
            pl.BlockSpec((1, 2, CHUNK, 4 * h), lambda i, j: (i, 0, 0, 0)),
            pl.BlockSpec((1, 2, CHUNK, 4 * h), lambda i, j: (i, 0, 0, 0)),
            pl.BlockSpec((1, 8, 4 * h), lambda i, j: (i, 0, 0)),
            pl.BlockSpec((1, nb, h), lambda i, j: (i, j, 1)),
        ],
        out_specs=pl.BlockSpec((1, nb, 3 * h), lambda i, j: (i, j, 0)),
        out_shape=jax.ShapeDtypeStruct((n_layers, n, 3 * h), jnp.float32),
    )(agg6, agg6, wxc, whc, bsum8, state)
    return out


def _sc_aggregate_all(xflat, stateflat, srcx, dstx, srch, dsth,
                      n_nodes, n_chunks_w, n_layers):
    rows_per_tile = (-(-(n_nodes + 1) // NS) + 7) // 8 * 8
    acc_rows = rows_per_tile * NS
    npass = 1 + n_layers

    mesh = plsc.VectorSubcoreMesh(core_axis_name="c", subcore_axis_name="s",
                                  num_cores=NC, num_subcores=NS)

    @functools.partial(
        pl.kernel,
        out_type=jax.ShapeDtypeStruct((npass * NC * acc_rows, CHUNK), jnp.float32),
        mesh=mesh,
        scratch_types=[
            pltpu.VMEM((32, CHUNK), jnp.float32),
            pltpu.VMEM((n_chunks_w, CHUNK), jnp.int32),
            pltpu.VMEM((n_chunks_w, CHUNK), jnp.int32),
            pltpu.VMEM((CHUNK, CHUNK), jnp.float32),
            pltpu.VMEM_SHARED((acc_rows, CHUNK), jnp.float32),
            pltpu.SemaphoreType.DMA,
        ],
    )
    def agg_kernel(x_hbm, st_hbm, srcx_hbm, dstx_hbm, srch_hbm, dsth_hbm, out_hbm,
                   zbuf, dstb, idxb, rowsb, acc, sem):
        c = lax.axis_index("c")
        s = lax.axis_index("s")
        w = s
        zero16 = jnp.zeros((16,), jnp.float32)

        def zloop(r, _):
            for k in range(CHUNK // 16):
                zbuf[r, pl.ds(k * 16, 16)] = zero16
            return 0
        lax.fori_loop(0, 32, zloop, 0)

        for p in range(npass):
            src_hbm = srcx_hbm if p == 0 else srch_hbm
            dst_hbm = dstx_hbm if p == 0 else dsth_hbm
            pltpu.sync_copy(src_hbm.at[pl.ds(w * n_chunks_w, n_chunks_w)], idxb)
            if p <= 1:
                pltpu.sync_copy(dst_hbm.at[pl.ds(w * n_chunks_w, n_chunks_w)], dstb)

            off = 0
            while off < rows_per_tile:
                step = min(32, rows_per_tile - off)
                pltpu.sync_copy(zbuf.at[pl.ds(0, step)],
                                acc.at[pl.ds(s * rows_per_tile + off, step)])
                off += step
            plsc.subcore_barrier()

            if p == 0:
                scale, base = 2, c
            else:
                scale, base = 6, (p - 1) * (n_nodes * 6) + c

            def tloop(j, _):
                for k in range(CHUNK // 16):
                    v = idxb[j, pl.ds(k * 16, 16)]
                    idxb[j, pl.ds(k * 16, 16)] = v * scale + base
                return 0
            lax.fori_loop(0, n_chunks_w, tloop, 0)

            src_ref = x_hbm if p == 0 else st_hbm

            def eloop(j, _):
                pltpu.async_copy(src_ref.at[idxb.at[j]], rowsb, sem).wait()
                pltpu.sync_copy(rowsb, acc.at[dstb.at[j]], add=True)
                return 0
            lax.fori_loop(0, n_chunks_w, eloop, 0)
            plsc.subcore_barrier()

            obase = (p * NC + c) * acc_rows + s * rows_per_tile
            pltpu.sync_copy(acc.at[pl.ds(s * rows_per_tile, rows_per_tile)],
                            out_hbm.at[pl.ds(obase, rows_per_tile)])
            plsc.subcore_barrier()

    return agg_kernel(xflat, stateflat, srcx, dstx, srch, dsth)


# speedup vs baseline: 13.3877x; 1.2054x over previous
import functools

import jax
import jax.numpy as jnp
from jax import lax
from jax.experimental import pallas as pl
from jax.experimental.pallas import tpu as pltpu
from jax.experimental.pallas import tpu_sc as plsc

NC = 2
NS = 16
CHUNK = 128


def _dense_body(aggx_ref, aggh_ref, wx_ref, wh_ref, b_ref, c_ref, out_ref):
    g = b_ref[0, 0:1, :].astype(jnp.float32)
    g = g + jnp.dot(aggx_ref[0], wx_ref[0, 0], preferred_element_type=jnp.float32)
    g = g + jnp.dot(aggx_ref[1], wx_ref[0, 1], preferred_element_type=jnp.float32)
    g = g + jnp.dot(aggh_ref[0], wh_ref[0, 0], preferred_element_type=jnp.float32)
    g = g + jnp.dot(aggh_ref[1], wh_ref[0, 1], preferred_element_type=jnp.float32)
    H = 256
    f_t = jax.nn.sigmoid(g[:, 0:H])
    i_t = jax.nn.sigmoid(g[:, H:2 * H])
    o_t = jax.nn.sigmoid(g[:, 2 * H:3 * H])
    z_t = jnp.tanh(g[:, 3 * H:4 * H])
    c_prev = c_ref[0]
    c_t = f_t * c_prev + i_t * z_t
    h_t = o_t * c_t
    out_ref[0] = jnp.concatenate([h_t, c_t, f_t], axis=-1)


def kernel(x, edge_index, state, prev_edge, Wx, bx, Wh, bh):
    n, d = x.shape
    n_layers = Wx.shape[0]
    h = Wx.shape[3]
    e = edge_index.shape[1]

    per_w_chunks = (-(-e // (NS * CHUNK)) + 7) // 8 * 8
    e_pad = NS * per_w_chunks * CHUNK
    pad = e_pad - e

    def pad_edges(ei):
        src = jnp.concatenate([ei[0], jnp.zeros((pad,), jnp.int32)])
        dst = jnp.concatenate([ei[1], jnp.full((pad,), n, jnp.int32)])
        return src.reshape(e_pad // CHUNK, CHUNK), dst.reshape(e_pad // CHUNK, CHUNK)

    srcx, dstx = pad_edges(edge_index)
    srch, dsth = pad_edges(prev_edge)

    xflat = x.reshape(n * (d // CHUNK), CHUNK)
    stateflat = state.reshape(n_layers * n * (state.shape[2] // CHUNK), CHUNK)

    agg_flat = _sc_aggregate_all(xflat, stateflat, srcx, dstx, srch, dsth,
                                 n, per_w_chunks, n_layers)
    region = (-(-(n + 1) // NS) + 7) // 8 * 8 * NS
    agg6 = agg_flat.reshape((1 + n_layers) * NC, region, CHUNK)

    wxc = jnp.transpose(Wx, (0, 2, 1, 3)).reshape(n_layers, d, 4 * h)
    wxc = wxc.reshape(n_layers, d // CHUNK, CHUNK, 4 * h)
    whc = jnp.transpose(Wh, (0, 2, 1, 3)).reshape(n_layers, h, 4 * h)
    whc = whc.reshape(n_layers, h // CHUNK, CHUNK, 4 * h)
    bsum = (bx + bh).reshape(n_layers, 4 * h)
    bsum8 = jnp.broadcast_to(bsum[:, None, :], (n_layers, 8, 4 * h))

    nb = 1000
    grid = (n_layers, n // nb)
    out = pl.pallas_call(
        _dense_body,
        grid=grid,
        in_specs=[
            pl.BlockSpec((2, nb, CHUNK), lambda i, j: (0, j, 0)),
            pl.BlockSpec((2, nb, CHUNK), lambda i, j: (1 + i, j, 0)),
            pl.BlockSpec((1, 2, CHUNK, 4 * h), lambda i, j: (i, 0, 0, 0)),
            pl.BlockSpec((1, 2, CHUNK, 4 * h), lambda i, j: (i, 0, 0, 0)),
            pl.BlockSpec((1, 8, 4 * h), lambda i, j: (i, 0, 0)),
            pl.BlockSpec((1, nb, h), lambda i, j: (i, j, 1)),
        ],
        out_specs=pl.BlockSpec((1, nb, 3 * h), lambda i, j: (i, j, 0)),
        out_shape=jax.ShapeDtypeStruct((n_layers, n, 3 * h), jnp.float32),
    )(agg6, agg6, wxc, whc, bsum8, state)
    return out


def _sc_aggregate_all(xflat, stateflat, srcx, dstx, srch, dsth,
                      n_nodes, n_chunks_w, n_layers):
    rows_per_tile = (-(-(n_nodes + 1) // NS) + 7) // 8 * 8
    acc_rows = rows_per_tile * NS
    npass = 1 + n_layers

    mesh = plsc.VectorSubcoreMesh(core_axis_name="c", subcore_axis_name="s",
                                  num_cores=NC, num_subcores=NS)

    assert n_chunks_w % 2 == 0
    nhalf = n_chunks_w // 2

    @functools.partial(
        pl.kernel,
        out_type=jax.ShapeDtypeStruct((npass * NC * acc_rows, CHUNK), jnp.float32),
        mesh=mesh,
        scratch_types=[
            pltpu.VMEM((8, CHUNK), jnp.float32),
            pltpu.VMEM((nhalf, CHUNK), jnp.int32),
            pltpu.VMEM((nhalf, CHUNK), jnp.int32),
            pltpu.VMEM((CHUNK, CHUNK), jnp.float32),
            pltpu.VMEM((CHUNK, CHUNK), jnp.float32),
            pltpu.VMEM_SHARED((acc_rows, CHUNK), jnp.float32),
            pltpu.SemaphoreType.DMA,
            pltpu.SemaphoreType.DMA,
        ],
    )
    def agg_kernel(x_hbm, st_hbm, srcx_hbm, dstx_hbm, srch_hbm, dsth_hbm, out_hbm,
                   zbuf, dstb, idxb, rows0, rows1, acc, sem0, sem1):
        c = lax.axis_index("c")
        s = lax.axis_index("s")
        w = s
        zero16 = jnp.zeros((16,), jnp.float32)
        rows = (rows0, rows1)
        sems = (sem0, sem1)

        def zloop(r, _):
            for k in range(CHUNK // 16):
                zbuf[r, pl.ds(k * 16, 16)] = zero16
            return 0
        lax.fori_loop(0, 8, zloop, 0)

        for p in range(npass):
            src_hbm = srcx_hbm if p == 0 else srch_hbm
            dst_hbm = dstx_hbm if p == 0 else dsth_hbm
            src_ref = x_hbm if p == 0 else st_hbm
            if p == 0:
                scale, base = 2, c
            else:
                scale, base = 6, (p - 1) * (n_nodes * 6) + c

            off = 0
            while off < rows_per_tile:
                step = min(8, rows_per_tile - off)
                pltpu.sync_copy(zbuf.at[pl.ds(0, step)],
                                acc.at[pl.ds(s * rows_per_tile + off, step)])
                off += step
            plsc.subcore_barrier()

            for hf in range(2):
                hbase = w * n_chunks_w + hf * nhalf
                pltpu.sync_copy(src_hbm.at[pl.ds(hbase, nhalf)], idxb)
                pltpu.sync_copy(dst_hbm.at[pl.ds(hbase, nhalf)], dstb)

                def tloop(j, _):
                    for k in range(CHUNK // 16):
                        v = idxb[j, pl.ds(k * 16, 16)]
                        idxb[j, pl.ds(k * 16, 16)] = v * scale + base
                    return 0
                lax.fori_loop(0, nhalf, tloop, 0)

                def start(j, b):
                    return pltpu.async_copy(src_ref.at[idxb.at[j]], rows[b],
                                            sems[b])
                def finish(j, b):
                    pltpu.make_async_copy(src_ref.at[idxb.at[j]], rows[b],
                                          sems[b]).wait()
                    pltpu.sync_copy(rows[b], acc.at[dstb.at[j]], add=True)

                start(0, 0)

                @pl.loop(0, nhalf - 2, step=2)
                def _(g):
                    start(g + 1, 1)
                    finish(g, 0)
                    start(g + 2, 0)
                    finish(g + 1, 1)

                start(nhalf - 1, 1)
                finish(nhalf - 2, 0)
                finish(nhalf - 1, 1)
            plsc.subcore_barrier()

            obase = (p * NC + c) * acc_rows + s * rows_per_tile
            pltpu.sync_copy(acc.at[pl.ds(s * rows_per_tile, rows_per_tile)],
                            out_hbm.at[pl.ds(obase, rows_per_tile)])
            plsc.subcore_barrier()

    return agg_kernel(xflat, stateflat, srcx, dstx, srch, dsth)


# speedup vs baseline: 13.7111x; 1.0242x over previous
import functools

import jax
import jax.numpy as jnp
from jax import lax
from jax.experimental import pallas as pl
from jax.experimental.pallas import tpu as pltpu
from jax.experimental.pallas import tpu_sc as plsc

NC = 2
NS = 16
CHUNK = 128


def _dense_body(aggx_ref, aggh_ref, wx_ref, wh_ref, b_ref, c_ref, out_ref):
    g = b_ref[0, 0:1, :].astype(jnp.float32)
    g = g + jnp.dot(aggx_ref[0], wx_ref[0, 0], preferred_element_type=jnp.float32)
    g = g + jnp.dot(aggx_ref[1], wx_ref[0, 1], preferred_element_type=jnp.float32)
    g = g + jnp.dot(aggh_ref[0], wh_ref[0, 0], preferred_element_type=jnp.float32)
    g = g + jnp.dot(aggh_ref[1], wh_ref[0, 1], preferred_element_type=jnp.float32)
    H = 256
    f_t = jax.nn.sigmoid(g[:, 0:H])
    i_t = jax.nn.sigmoid(g[:, H:2 * H])
    o_t = jax.nn.sigmoid(g[:, 2 * H:3 * H])
    z_t = jnp.tanh(g[:, 3 * H:4 * H])
    c_prev = c_ref[0]
    c_t = f_t * c_prev + i_t * z_t
    h_t = o_t * c_t
    out_ref[0] = jnp.concatenate([h_t, c_t, f_t], axis=-1)


def kernel(x, edge_index, state, prev_edge, Wx, bx, Wh, bh):
    n, d = x.shape
    n_layers = Wx.shape[0]
    h = Wx.shape[3]
    e = edge_index.shape[1]

    per_w_chunks = (-(-e // (NS * CHUNK)) + 7) // 8 * 8
    e_pad = NS * per_w_chunks * CHUNK
    pad = e_pad - e

    def pad_edges(ei):
        src = jnp.concatenate([ei[0], jnp.zeros((pad,), jnp.int32)])
        dst = jnp.concatenate([ei[1], jnp.full((pad,), n, jnp.int32)])
        return src.reshape(e_pad // CHUNK, CHUNK), dst.reshape(e_pad // CHUNK, CHUNK)

    srcx, dstx = pad_edges(edge_index)
    srch, dsth = pad_edges(prev_edge)

    xflat = x.reshape(n * (d // CHUNK), CHUNK)
    stateflat = state[:, :, :h].reshape(n_layers * n * (h // CHUNK), CHUNK)

    agg_flat = _sc_aggregate_all(xflat, stateflat, srcx, dstx, srch, dsth,
                                 n, per_w_chunks, n_layers)
    region = (-(-(n + 1) // NS) + 7) // 8 * 8 * NS
    agg6 = agg_flat.reshape((1 + n_layers) * NC, region, CHUNK)

    wxc = jnp.transpose(Wx, (0, 2, 1, 3)).reshape(n_layers, d, 4 * h)
    wxc = wxc.reshape(n_layers, d // CHUNK, CHUNK, 4 * h)
    whc = jnp.transpose(Wh, (0, 2, 1, 3)).reshape(n_layers, h, 4 * h)
    whc = whc.reshape(n_layers, h // CHUNK, CHUNK, 4 * h)
    bsum = (bx + bh).reshape(n_layers, 4 * h)
    bsum8 = jnp.broadcast_to(bsum[:, None, :], (n_layers, 8, 4 * h))

    nb = 1000
    grid = (n_layers, n // nb)
    out = pl.pallas_call(
        _dense_body,
        grid=grid,
        in_specs=[
            pl.BlockSpec((2, nb, CHUNK), lambda i, j: (0, j, 0)),
            pl.BlockSpec((2, nb, CHUNK), lambda i, j: (1 + i, j, 0)),
            pl.BlockSpec((1, 2, CHUNK, 4 * h), lambda i, j: (i, 0, 0, 0)),
            pl.BlockSpec((1, 2, CHUNK, 4 * h), lambda i, j: (i, 0, 0, 0)),
            pl.BlockSpec((1, 8, 4 * h), lambda i, j: (i, 0, 0)),
            pl.BlockSpec((1, nb, h), lambda i, j: (i, j, 1)),
        ],
        out_specs=pl.BlockSpec((1, nb, 3 * h), lambda i, j: (i, j, 0)),
        out_shape=jax.ShapeDtypeStruct((n_layers, n, 3 * h), jnp.float32),
    )(agg6, agg6, wxc, whc, bsum8, state)
    return out


def _sc_aggregate_all(xflat, stateflat, srcx, dstx, srch, dsth,
                      n_nodes, n_chunks_w, n_layers):
    rows_per_tile = (-(-(n_nodes + 1) // NS) + 7) // 8 * 8
    acc_rows = rows_per_tile * NS
    npass = 1 + n_layers

    mesh = plsc.VectorSubcoreMesh(core_axis_name="c", subcore_axis_name="s",
                                  num_cores=NC, num_subcores=NS)

    assert n_chunks_w % 2 == 0
    nhalf = n_chunks_w // 2

    @functools.partial(
        pl.kernel,
        out_type=jax.ShapeDtypeStruct((npass * NC * acc_rows, CHUNK), jnp.float32),
        mesh=mesh,
        scratch_types=[
            pltpu.VMEM((8, CHUNK), jnp.float32),
            pltpu.VMEM((nhalf, CHUNK), jnp.int32),
            pltpu.VMEM((nhalf, CHUNK), jnp.int32),
            pltpu.VMEM((CHUNK, CHUNK), jnp.float32),
            pltpu.VMEM((CHUNK, CHUNK), jnp.float32),
            pltpu.VMEM_SHARED((acc_rows, CHUNK), jnp.float32),
            pltpu.SemaphoreType.DMA,
            pltpu.SemaphoreType.DMA,
        ],
    )
    def agg_kernel(x_hbm, st_hbm, srcx_hbm, dstx_hbm, srch_hbm, dsth_hbm, out_hbm,
                   zbuf, dstb, idxb, rows0, rows1, acc, sem0, sem1):
        c = lax.axis_index("c")
        s = lax.axis_index("s")
        w = s
        zero16 = jnp.zeros((16,), jnp.float32)
        rows = (rows0, rows1)
        sems = (sem0, sem1)

        def zloop(r, _):
            for k in range(CHUNK // 16):
                zbuf[r, pl.ds(k * 16, 16)] = zero16
            return 0
        lax.fori_loop(0, 8, zloop, 0)

        for p in range(npass):
            src_hbm = srcx_hbm if p == 0 else srch_hbm
            dst_hbm = dstx_hbm if p == 0 else dsth_hbm
            src_ref = x_hbm if p == 0 else st_hbm
            if p == 0:
                scale, base = 2, c
            else:
                scale, base = 2, (p - 1) * (n_nodes * 2) + c

            off = 0
            while off < rows_per_tile:
                step = min(8, rows_per_tile - off)
                pltpu.sync_copy(zbuf.at[pl.ds(0, step)],
                                acc.at[pl.ds(s * rows_per_tile + off, step)])
                off += step
            plsc.subcore_barrier()

            for hf in range(2):
                hbase = w * n_chunks_w + hf * nhalf
                pltpu.sync_copy(src_hbm.at[pl.ds(hbase, nhalf)], idxb)
                pltpu.sync_copy(dst_hbm.at[pl.ds(hbase, nhalf)], dstb)

                def tloop(j, _):
                    for k in range(CHUNK // 16):
                        v = idxb[j, pl.ds(k * 16, 16)]
                        idxb[j, pl.ds(k * 16, 16)] = v * scale + base
                    return 0
                lax.fori_loop(0, nhalf, tloop, 0)

                def start(j, b):
                    return pltpu.async_copy(src_ref.at[idxb.at[j]], rows[b],
                                            sems[b])
                def finish(j, b):
                    pltpu.make_async_copy(src_ref.at[idxb.at[j]], rows[b],
                                          sems[b]).wait()
                    pltpu.sync_copy(rows[b], acc.at[dstb.at[j]], add=True)

                start(0, 0)

                @pl.loop(0, nhalf - 2, step=2)
                def _(g):
                    start(g + 1, 1)
                    finish(g, 0)
                    start(g + 2, 0)
                    finish(g + 1, 1)

                start(nhalf - 1, 1)
                finish(nhalf - 2, 0)
                finish(nhalf - 1, 1)
            plsc.subcore_barrier()

            obase = (p * NC + c) * acc_rows + s * rows_per_tile
            pltpu.sync_copy(acc.at[pl.ds(s * rows_per_tile, rows_per_tile)],
                            out_hbm.at[pl.ds(obase, rows_per_tile)])
            plsc.subcore_barrier()

    return agg_kernel(xflat, stateflat, srcx, dstx, srch, dsth)


# speedup vs baseline: 14.5820x; 1.0635x over previous
import functools

import jax
import jax.numpy as jnp
from jax import lax
from jax.experimental import pallas as pl
from jax.experimental.pallas import tpu as pltpu
from jax.experimental.pallas import tpu_sc as plsc

NC = 2
NS = 16
CHUNK = 128


def _dense_body(aggx_ref, aggh_ref, wx_ref, wh_ref, b_ref, c_ref, out_ref):
    g = b_ref[0, 0:1, :].astype(jnp.float32)
    g = g + jnp.dot(aggx_ref[0], wx_ref[0, 0], preferred_element_type=jnp.float32)
    g = g + jnp.dot(aggx_ref[1], wx_ref[0, 1], preferred_element_type=jnp.float32)
    g = g + jnp.dot(aggh_ref[0], wh_ref[0, 0], preferred_element_type=jnp.float32)
    g = g + jnp.dot(aggh_ref[1], wh_ref[0, 1], preferred_element_type=jnp.float32)
    H = 256
    f_t = jax.nn.sigmoid(g[:, 0:H])
    i_t = jax.nn.sigmoid(g[:, H:2 * H])
    o_t = jax.nn.sigmoid(g[:, 2 * H:3 * H])
    z_t = jnp.tanh(g[:, 3 * H:4 * H])
    c_prev = c_ref[0]
    c_t = f_t * c_prev + i_t * z_t
    h_t = o_t * c_t
    out_ref[0] = jnp.concatenate([h_t, c_t, f_t], axis=-1)


def kernel(x, edge_index, state, prev_edge, Wx, bx, Wh, bh):
    n, d = x.shape
    n_layers = Wx.shape[0]
    h = Wx.shape[3]
    e = edge_index.shape[1]

    per_w_chunks = (-(-e // (NS * CHUNK)) + 7) // 8 * 8
    e_pad = NS * per_w_chunks * CHUNK
    pad = e_pad - e

    def pad_edges(ei):
        src = jnp.concatenate([ei[0], jnp.zeros((pad,), jnp.int32)])
        dst = jnp.concatenate([ei[1], jnp.full((pad,), n, jnp.int32)])
        return src.reshape(e_pad // CHUNK, CHUNK), dst.reshape(e_pad // CHUNK, CHUNK)

    srcx, dstx = pad_edges(edge_index)
    srch, dsth = pad_edges(prev_edge)

    xflat = jnp.transpose(x.reshape(n, d // CHUNK, CHUNK), (1, 0, 2))
    xflat = xflat.reshape(n * (d // CHUNK), CHUNK)
    hpart = state[:, :, :h].reshape(n_layers, n, h // CHUNK, CHUNK)
    stateflat = jnp.transpose(hpart, (0, 2, 1, 3)).reshape(
        n_layers * (h // CHUNK) * n, CHUNK)

    agg_flat = _sc_aggregate_all(xflat, stateflat, srcx, dstx, srch, dsth,
                                 n, per_w_chunks, n_layers)
    region = (-(-(n + 1) // NS) + 7) // 8 * 8 * NS
    agg6 = agg_flat.reshape((1 + n_layers) * NC, region, CHUNK)

    wxc = jnp.transpose(Wx, (0, 2, 1, 3)).reshape(n_layers, d, 4 * h)
    wxc = wxc.reshape(n_layers, d // CHUNK, CHUNK, 4 * h)
    whc = jnp.transpose(Wh, (0, 2, 1, 3)).reshape(n_layers, h, 4 * h)
    whc = whc.reshape(n_layers, h // CHUNK, CHUNK, 4 * h)
    bsum = (bx + bh).reshape(n_layers, 4 * h)
    bsum8 = jnp.broadcast_to(bsum[:, None, :], (n_layers, 8, 4 * h))

    nb = 1000
    grid = (n_layers, n // nb)
    out = pl.pallas_call(
        _dense_body,
        grid=grid,
        in_specs=[
            pl.BlockSpec((2, nb, CHUNK), lambda i, j: (0, j, 0)),
            pl.BlockSpec((2, nb, CHUNK), lambda i, j: (1 + i, j, 0)),
            pl.BlockSpec((1, 2, CHUNK, 4 * h), lambda i, j: (i, 0, 0, 0)),
            pl.BlockSpec((1, 2, CHUNK, 4 * h), lambda i, j: (i, 0, 0, 0)),
            pl.BlockSpec((1, 8, 4 * h), lambda i, j: (i, 0, 0)),
            pl.BlockSpec((1, nb, h), lambda i, j: (i, j, 1)),
        ],
        out_specs=pl.BlockSpec((1, nb, 3 * h), lambda i, j: (i, j, 0)),
        out_shape=jax.ShapeDtypeStruct((n_layers, n, 3 * h), jnp.float32),
    )(agg6, agg6, wxc, whc, bsum8, state)
    return out


def _sc_aggregate_all(xflat, stateflat, srcx, dstx, srch, dsth,
                      n_nodes, n_chunks_w, n_layers):
    rows_per_tile = (-(-(n_nodes + 1) // NS) + 7) // 8 * 8
    acc_rows = rows_per_tile * NS
    npass = 1 + n_layers

    mesh = plsc.VectorSubcoreMesh(core_axis_name="c", subcore_axis_name="s",
                                  num_cores=NC, num_subcores=NS)

    assert n_chunks_w % 2 == 0
    nhalf = n_chunks_w // 2

    @functools.partial(
        pl.kernel,
        out_type=jax.ShapeDtypeStruct((npass * NC * acc_rows, CHUNK), jnp.float32),
        mesh=mesh,
        scratch_types=[
            pltpu.VMEM((8, CHUNK), jnp.float32),
            pltpu.VMEM((nhalf, CHUNK), jnp.int32),
            pltpu.VMEM((nhalf, CHUNK), jnp.int32),
            pltpu.VMEM((CHUNK, CHUNK), jnp.float32),
            pltpu.VMEM((CHUNK, CHUNK), jnp.float32),
            pltpu.VMEM_SHARED((acc_rows, CHUNK), jnp.float32),
            pltpu.SemaphoreType.DMA,
            pltpu.SemaphoreType.DMA,
        ],
    )
    def agg_kernel(x_hbm, st_hbm, srcx_hbm, dstx_hbm, srch_hbm, dsth_hbm, out_hbm,
                   zbuf, dstb, idxb, rows0, rows1, acc, sem0, sem1):
        c = lax.axis_index("c")
        s = lax.axis_index("s")
        w = s
        zero16 = jnp.zeros((16,), jnp.float32)
        rows = (rows0, rows1)
        sems = (sem0, sem1)

        def zloop(r, _):
            for k in range(CHUNK // 16):
                zbuf[r, pl.ds(k * 16, 16)] = zero16
            return 0
        lax.fori_loop(0, 8, zloop, 0)

        for p in range(npass):
            src_hbm = srcx_hbm if p == 0 else srch_hbm
            dst_hbm = dstx_hbm if p == 0 else dsth_hbm
            src_ref = x_hbm if p == 0 else st_hbm
            if p == 0:
                base = c * n_nodes
            else:
                base = ((p - 1) * 2 + c) * n_nodes

            off = 0
            while off < rows_per_tile:
                step = min(8, rows_per_tile - off)
                pltpu.sync_copy(zbuf.at[pl.ds(0, step)],
                                acc.at[pl.ds(s * rows_per_tile + off, step)])
                off += step
            plsc.subcore_barrier()

            for hf in range(2):
                hbase = w * n_chunks_w + hf * nhalf
                pltpu.sync_copy(src_hbm.at[pl.ds(hbase, nhalf)], idxb)
                pltpu.sync_copy(dst_hbm.at[pl.ds(hbase, nhalf)], dstb)

                def tloop(j, _):
                    for k in range(CHUNK // 16):
                        v = idxb[j, pl.ds(k * 16, 16)]
                        idxb[j, pl.ds(k * 16, 16)] = v + base
                    return 0
                lax.fori_loop(0, nhalf, tloop, 0)

                def start(j, b):
                    return pltpu.async_copy(src_ref.at[idxb.at[j]], rows[b],
                                            sems[b])
                def finish(j, b):
                    pltpu.make_async_copy(src_ref.at[idxb.at[j]], rows[b],
                                          sems[b]).wait()
                    pltpu.sync_copy(rows[b], acc.at[dstb.at[j]], add=True)

                start(0, 0)

                @pl.loop(0, nhalf - 2, step=2)
                def _(g):
                    start(g + 1, 1)
                    finish(g, 0)
                    start(g + 2, 0)
                    finish(g + 1, 1)

                start(nhalf - 1, 1)
                finish(nhalf - 2, 0)
                finish(nhalf - 1, 1)
            plsc.subcore_barrier()

            obase = (p * NC + c) * acc_rows + s * rows_per_tile
            pltpu.sync_copy(acc.at[pl.ds(s * rows_per_tile, rows_per_tile)],
                            out_hbm.at[pl.ds(obase, rows_per_tile)])
            plsc.subcore_barrier()

    return agg_kernel(xflat, stateflat, srcx, dstx, srch, dsth)


# speedup vs baseline: 14.6973x; 1.0079x over previous
import functools

import jax
import jax.numpy as jnp
from jax import lax
from jax.experimental import pallas as pl
from jax.experimental.pallas import tpu as pltpu
from jax.experimental.pallas import tpu_sc as plsc

NC = 2
NS = 16
CHUNK = 128


def _dense_body(aggx_ref, aggh_ref, w_ref, b_ref, c_ref, out_ref):
    a_cat = jnp.concatenate(
        [aggx_ref[0], aggx_ref[1], aggh_ref[0], aggh_ref[1]], axis=1)
    g = b_ref[0, 0:1, :].astype(jnp.float32)
    g = g + jnp.dot(a_cat, w_ref[0], preferred_element_type=jnp.float32)
    H = 256
    f_t = jax.nn.sigmoid(g[:, 0:H])
    i_t = jax.nn.sigmoid(g[:, H:2 * H])
    o_t = jax.nn.sigmoid(g[:, 2 * H:3 * H])
    z_t = jnp.tanh(g[:, 3 * H:4 * H])
    c_prev = c_ref[0]
    c_t = f_t * c_prev + i_t * z_t
    h_t = o_t * c_t
    out_ref[0] = jnp.concatenate([h_t, c_t, f_t], axis=-1)


def kernel(x, edge_index, state, prev_edge, Wx, bx, Wh, bh):
    n, d = x.shape
    n_layers = Wx.shape[0]
    h = Wx.shape[3]
    e = edge_index.shape[1]

    per_w_chunks = (-(-e // (NS * CHUNK)) + 7) // 8 * 8
    e_pad = NS * per_w_chunks * CHUNK
    pad = e_pad - e

    def pad_edges(ei):
        src = jnp.concatenate([ei[0], jnp.zeros((pad,), jnp.int32)])
        dst = jnp.concatenate([ei[1], jnp.full((pad,), n, jnp.int32)])
        return src.reshape(e_pad // CHUNK, CHUNK), dst.reshape(e_pad // CHUNK, CHUNK)

    srcx, dstx = pad_edges(edge_index)
    srch, dsth = pad_edges(prev_edge)

    xflat = jnp.transpose(x.reshape(n, d // CHUNK, CHUNK), (1, 0, 2))
    xflat = xflat.reshape(n * (d // CHUNK), CHUNK)
    hpart = state[:, :, :h].reshape(n_layers, n, h // CHUNK, CHUNK)
    stateflat = jnp.transpose(hpart, (0, 2, 1, 3)).reshape(
        n_layers * (h // CHUNK) * n, CHUNK)

    agg_flat = _sc_aggregate_all(xflat, stateflat, srcx, dstx, srch, dsth,
                                 n, per_w_chunks, n_layers)
    region = (-(-(n + 1) // NS) + 7) // 8 * 8 * NS
    agg6 = agg_flat.reshape((1 + n_layers) * NC, region, CHUNK)

    wxc = jnp.transpose(Wx, (0, 2, 1, 3)).reshape(n_layers, d, 4 * h)
    whc = jnp.transpose(Wh, (0, 2, 1, 3)).reshape(n_layers, h, 4 * h)
    wcat = jnp.concatenate([wxc, whc], axis=1)
    bsum = (bx + bh).reshape(n_layers, 4 * h)
    bsum8 = jnp.broadcast_to(bsum[:, None, :], (n_layers, 8, 4 * h))

    nb = 1000
    grid = (n_layers, n // nb)
    out = pl.pallas_call(
        _dense_body,
        grid=grid,
        in_specs=[
            pl.BlockSpec((2, nb, CHUNK), lambda i, j: (0, j, 0)),
            pl.BlockSpec((2, nb, CHUNK), lambda i, j: (1 + i, j, 0)),
            pl.BlockSpec((1, d + h, 4 * h), lambda i, j: (i, 0, 0)),
            pl.BlockSpec((1, 8, 4 * h), lambda i, j: (i, 0, 0)),
            pl.BlockSpec((1, nb, h), lambda i, j: (i, j, 1)),
        ],
        out_specs=pl.BlockSpec((1, nb, 3 * h), lambda i, j: (i, j, 0)),
        out_shape=jax.ShapeDtypeStruct((n_layers, n, 3 * h), jnp.float32),
    )(agg6, agg6, wcat, bsum8, state)
    return out


def _sc_aggregate_all(xflat, stateflat, srcx, dstx, srch, dsth,
                      n_nodes, n_chunks_w, n_layers):
    rows_per_tile = (-(-(n_nodes + 1) // NS) + 7) // 8 * 8
    acc_rows = rows_per_tile * NS
    npass = 1 + n_layers

    mesh = plsc.VectorSubcoreMesh(core_axis_name="c", subcore_axis_name="s",
                                  num_cores=NC, num_subcores=NS)

    assert n_chunks_w % 2 == 0
    nhalf = n_chunks_w // 2

    @functools.partial(
        pl.kernel,
        out_type=jax.ShapeDtypeStruct((npass * NC * acc_rows, CHUNK), jnp.float32),
        mesh=mesh,
        scratch_types=[
            pltpu.VMEM((8, CHUNK), jnp.float32),
            pltpu.VMEM((nhalf, CHUNK), jnp.int32),
            pltpu.VMEM((nhalf, CHUNK), jnp.int32),
            pltpu.VMEM((CHUNK, CHUNK), jnp.float32),
            pltpu.VMEM((CHUNK, CHUNK), jnp.float32),
            pltpu.VMEM_SHARED((acc_rows, CHUNK), jnp.float32),
            pltpu.SemaphoreType.DMA,
            pltpu.SemaphoreType.DMA,
        ],
    )
    def agg_kernel(x_hbm, st_hbm, srcx_hbm, dstx_hbm, srch_hbm, dsth_hbm, out_hbm,
                   zbuf, dstb, idxb, rows0, rows1, acc, sem0, sem1):
        c = lax.axis_index("c")
        s = lax.axis_index("s")
        w = s
        zero16 = jnp.zeros((16,), jnp.float32)
        rows = (rows0, rows1)
        sems = (sem0, sem1)

        def zloop(r, _):
            for k in range(CHUNK // 16):
                zbuf[r, pl.ds(k * 16, 16)] = zero16
            return 0
        lax.fori_loop(0, 8, zloop, 0)

        for p in range(npass):
            src_hbm = srcx_hbm if p == 0 else srch_hbm
            dst_hbm = dstx_hbm if p == 0 else dsth_hbm
            src_ref = x_hbm if p == 0 else st_hbm
            if p == 0:
                base = c * n_nodes
            else:
                base = ((p - 1) * 2 + c) * n_nodes

            off = 0
            while off < rows_per_tile:
                step = min(8, rows_per_tile - off)
                pltpu.sync_copy(zbuf.at[pl.ds(0, step)],
                                acc.at[pl.ds(s * rows_per_tile + off, step)])
                off += step
            plsc.subcore_barrier()

            for hf in range(2):
                hbase = w * n_chunks_w + hf * nhalf
                pltpu.sync_copy(src_hbm.at[pl.ds(hbase, nhalf)], idxb)
                pltpu.sync_copy(dst_hbm.at[pl.ds(hbase, nhalf)], dstb)

                def tloop(j, _):
                    for k in range(CHUNK // 16):
                        v = idxb[j, pl.ds(k * 16, 16)]
                        idxb[j, pl.ds(k * 16, 16)] = v + base
                    return 0
                lax.fori_loop(0, nhalf, tloop, 0)

                def start(j, b):
                    return pltpu.async_copy(src_ref.at[idxb.at[j]], rows[b],
                                            sems[b])
                def finish(j, b):
                    pltpu.make_async_copy(src_ref.at[idxb.at[j]], rows[b],
                                          sems[b]).wait()
                    pltpu.sync_copy(rows[b], acc.at[dstb.at[j]], add=True)

                start(0, 0)

                @pl.loop(0, nhalf - 2, step=2)
                def _(g):
                    start(g + 1, 1)
                    finish(g, 0)
                    start(g + 2, 0)
                    finish(g + 1, 1)

                start(nhalf - 1, 1)
                finish(nhalf - 2, 0)
                finish(nhalf - 1, 1)
            plsc.subcore_barrier()

            obase = (p * NC + c) * acc_rows + s * rows_per_tile
            pltpu.sync_copy(acc.at[pl.ds(s * rows_per_tile, rows_per_tile)],
                            out_hbm.at[pl.ds(obase, rows_per_tile)])
            plsc.subcore_barrier()

    return agg_kernel(xflat, stateflat, srcx, dstx, srch, dsth)


# speedup vs baseline: 14.8370x; 1.0095x over previous
import functools

import jax
import jax.numpy as jnp
from jax import lax
from jax.experimental import pallas as pl
from jax.experimental.pallas import tpu as pltpu
from jax.experimental.pallas import tpu_sc as plsc

NC = 2
NS = 16
CHUNK = 128


def _dense_body(aggx_ref, aggh_ref, w_ref, b_ref, c_ref, out_ref):
    a_cat = jnp.concatenate(
        [aggx_ref[0], aggx_ref[1], aggh_ref[0], aggh_ref[1]], axis=1)
    g = b_ref[0, 0:1, :].astype(jnp.float32)
    g = g + jnp.dot(a_cat, w_ref[0], preferred_element_type=jnp.float32)
    H = 256
    f_t = jax.nn.sigmoid(g[:, 0:H])
    i_t = jax.nn.sigmoid(g[:, H:2 * H])
    o_t = jax.nn.sigmoid(g[:, 2 * H:3 * H])
    z_t = jnp.tanh(g[:, 3 * H:4 * H])
    c_prev = c_ref[0]
    c_t = f_t * c_prev + i_t * z_t
    h_t = o_t * c_t
    out_ref[0] = jnp.concatenate([h_t, c_t, f_t], axis=-1)


def kernel(x, edge_index, state, prev_edge, Wx, bx, Wh, bh):
    n, d = x.shape
    n_layers = Wx.shape[0]
    h = Wx.shape[3]
    e = edge_index.shape[1]

    per_w_chunks = (-(-e // (NS * CHUNK)) + 7) // 8 * 8
    e_pad = NS * per_w_chunks * CHUNK
    pad = e_pad - e

    def pad_edges(ei):
        src = jnp.concatenate([ei[0], jnp.zeros((pad,), jnp.int32)])
        dst = jnp.concatenate([ei[1], jnp.full((pad,), n, jnp.int32)])
        return src.reshape(e_pad // CHUNK, CHUNK), dst.reshape(e_pad // CHUNK, CHUNK)

    srcx, dstx = pad_edges(edge_index)
    srch, dsth = pad_edges(prev_edge)

    xflat = jnp.transpose(x.reshape(n, d // CHUNK, CHUNK), (1, 0, 2))
    xflat = xflat.reshape(n * (d // CHUNK), CHUNK)
    hpart = state[:, :, :h].reshape(n_layers, n, h // CHUNK, CHUNK)
    stateflat = jnp.transpose(hpart, (0, 2, 1, 3)).reshape(
        n_layers * (h // CHUNK) * n, CHUNK)

    agg_flat = _sc_aggregate_all(xflat, stateflat, srcx, dstx, srch, dsth,
                                 n, per_w_chunks, n_layers)
    region = (-(-(n + 1) // NS) + 7) // 8 * 8 * NS
    agg6 = agg_flat.reshape((1 + n_layers) * NC, region, CHUNK)

    wxc = jnp.transpose(Wx, (0, 2, 1, 3)).reshape(n_layers, d, 4 * h)
    whc = jnp.transpose(Wh, (0, 2, 1, 3)).reshape(n_layers, h, 4 * h)
    wcat = jnp.concatenate([wxc, whc], axis=1)
    bsum = (bx + bh).reshape(n_layers, 4 * h)
    bsum8 = jnp.broadcast_to(bsum[:, None, :], (n_layers, 8, 4 * h))

    nb = 2000
    grid = (n_layers, n // nb)
    out = pl.pallas_call(
        _dense_body,
        grid=grid,
        in_specs=[
            pl.BlockSpec((2, nb, CHUNK), lambda i, j: (0, j, 0)),
            pl.BlockSpec((2, nb, CHUNK), lambda i, j: (1 + i, j, 0)),
            pl.BlockSpec((1, d + h, 4 * h), lambda i, j: (i, 0, 0)),
            pl.BlockSpec((1, 8, 4 * h), lambda i, j: (i, 0, 0)),
            pl.BlockSpec((1, nb, h), lambda i, j: (i, j, 1)),
        ],
        out_specs=pl.BlockSpec((1, nb, 3 * h), lambda i, j: (i, j, 0)),
        out_shape=jax.ShapeDtypeStruct((n_layers, n, 3 * h), jnp.float32),
    )(agg6, agg6, wcat, bsum8, state)
    return out


def _sc_aggregate_all(xflat, stateflat, srcx, dstx, srch, dsth,
                      n_nodes, n_chunks_w, n_layers):
    rows_per_tile = (-(-(n_nodes + 1) // NS) + 7) // 8 * 8
    acc_rows = rows_per_tile * NS
    npass = 1 + n_layers

    mesh = plsc.VectorSubcoreMesh(core_axis_name="c", subcore_axis_name="s",
                                  num_cores=NC, num_subcores=NS)

    assert n_chunks_w % 2 == 0
    nhalf = n_chunks_w // 2

    @functools.partial(
        pl.kernel,
        out_type=jax.ShapeDtypeStruct((npass * NC * acc_rows, CHUNK), jnp.float32),
        mesh=mesh,
        scratch_types=[
            pltpu.VMEM((8, CHUNK), jnp.float32),
            pltpu.VMEM((nhalf, CHUNK), jnp.int32),
            pltpu.VMEM((nhalf, CHUNK), jnp.int32),
            pltpu.VMEM((CHUNK, CHUNK), jnp.float32),
            pltpu.VMEM((CHUNK, CHUNK), jnp.float32),
            pltpu.VMEM_SHARED((acc_rows, CHUNK), jnp.float32),
            pltpu.SemaphoreType.DMA,
            pltpu.SemaphoreType.DMA,
        ],
    )
    def agg_kernel(x_hbm, st_hbm, srcx_hbm, dstx_hbm, srch_hbm, dsth_hbm, out_hbm,
                   zbuf, dstb, idxb, rows0, rows1, acc, sem0, sem1):
        c = lax.axis_index("c")
        s = lax.axis_index("s")
        w = s
        zero16 = jnp.zeros((16,), jnp.float32)
        rows = (rows0, rows1)
        sems = (sem0, sem1)

        def zloop(r, _):
            for k in range(CHUNK // 16):
                zbuf[r, pl.ds(k * 16, 16)] = zero16
            return 0
        lax.fori_loop(0, 8, zloop, 0)

        for p in range(npass):
            src_hbm = srcx_hbm if p == 0 else srch_hbm
            dst_hbm = dstx_hbm if p == 0 else dsth_hbm
            src_ref = x_hbm if p == 0 else st_hbm
            if p == 0:
                base = c * n_nodes
            else:
                base = ((p - 1) * 2 + c) * n_nodes

            off = 0
            while off < rows_per_tile:
                step = min(8, rows_per_tile - off)
                pltpu.sync_copy(zbuf.at[pl.ds(0, step)],
                                acc.at[pl.ds(s * rows_per_tile + off, step)])
                off += step
            plsc.subcore_barrier()

            for hf in range(2):
                hbase = w * n_chunks_w + hf * nhalf
                pltpu.sync_copy(src_hbm.at[pl.ds(hbase, nhalf)], idxb)
                pltpu.sync_copy(dst_hbm.at[pl.ds(hbase, nhalf)], dstb)

                def tloop(j, _):
                    for k in range(CHUNK // 16):
                        v = idxb[j, pl.ds(k * 16, 16)]
                        idxb[j, pl.ds(k * 16, 16)] = v + base
                    return 0
                lax.fori_loop(0, nhalf, tloop, 0)

                def start(j, b):
                    return pltpu.async_copy(src_ref.at[idxb.at[j]], rows[b],
                                            sems[b])
                def finish(j, b):
                    pltpu.make_async_copy(src_ref.at[idxb.at[j]], rows[b],
                                          sems[b]).wait()
                    pltpu.sync_copy(rows[b], acc.at[dstb.at[j]], add=True)

                start(0, 0)

                @pl.loop(0, nhalf - 2, step=2)
                def _(g):
                    start(g + 1, 1)
                    finish(g, 0)
                    start(g + 2, 0)
                    finish(g + 1, 1)

                start(nhalf - 1, 1)
                finish(nhalf - 2, 0)
                finish(nhalf - 1, 1)
            plsc.subcore_barrier()

            obase = (p * NC + c) * acc_rows + s * rows_per_tile
            pltpu.sync_copy(acc.at[pl.ds(s * rows_per_tile, rows_per_tile)],
                            out_hbm.at[pl.ds(obase, rows_per_tile)])
            plsc.subcore_barrier()

    return agg_kernel(xflat, stateflat, srcx, dstx, srch, dsth)
